# Initial kernel scaffold; baseline (speedup 1.0000x reference)
import jax
import jax.numpy as jnp
from jax.experimental import pallas as pl


def kernel(x, x_edge, ln1_g, ln1_b, Wn, bn, We1, be1, We2, be2, root, cb, Wf1, bf1, Wf2, bf2, lno_g, lno_b):
    raise NotImplementedError("write your pallas kernel here")


# speedup vs baseline: 5.9089x; 5.9089x over previous
import functools
import numpy as np
import jax
import jax.numpy as jnp
from jax import lax
from jax.experimental import pallas as pl
from jax.experimental.pallas import tpu as pltpu
from jax.experimental.pallas import tpu_sc as plsc

N = 64
B = 64
H = 16
DIN = 128
DE = 4
EPG = 4032
NN = N * N
G = 8
NG = N // G
GH = NG // 2
RH = GH * N
KW = G * 64


def _build_perm():
    tri = np.zeros((N, N), dtype=np.int64)
    iu = np.triu_indices(N, k=1)
    tri[iu] = np.arange(len(iu[0]))
    i = np.arange(N)[:, None]
    j = np.arange(N)[None, :]
    perm2d = np.where(i < j, tri, np.where(i > j, 2016 + tri.T, 0))
    r = np.arange(NN)
    m = r % G
    t = r // G
    jj = t % N
    gg = t // N
    return perm2d[(G * gg + m), jj].astype(np.int32)


_PERM = _build_perm()
_PIDX = (_PERM[:, None].astype(np.int32) * DE + np.arange(DE, dtype=np.int32)[None, :]).reshape(-1)

_NW = 32
_GPW = B // _NW
_SRCW = EPG * DE
_OUTW = NN * DE
_L = 16


def _sc_permute(xe2d, pidx):
    mesh = plsc.VectorSubcoreMesh(core_axis_name="c", subcore_axis_name="s")

    @functools.partial(
        pl.kernel, mesh=mesh,
        out_type=jax.ShapeDtypeStruct((B, _OUTW), jnp.float32),
        compiler_params=pltpu.CompilerParams(use_tc_tiling_on_sc=False,
                                             needs_layout_passes=False),
        scratch_types=[
            pltpu.VMEM((_OUTW,), jnp.int32),
            pltpu.VMEM((_SRCW,), jnp.float32),
            pltpu.VMEM((_OUTW,), jnp.float32),
        ],
    )
    def gather_k(pidx_hbm, src_hbm, out_hbm, idx_sp, src_sp, out_sp):
        wid = lax.axis_index("s") * 2 + lax.axis_index("c")
        pltpu.sync_copy(pidx_hbm, idx_sp)
        for rep in range(_GPW):
            b = wid * _GPW + rep
            pltpu.sync_copy(src_hbm.at[b], src_sp)

            def step(t, carry):
                for u in range(8):
                    v = t * 8 + u
                    ivec = idx_sp[pl.ds(v * _L, _L)]
                    out_sp[pl.ds(v * _L, _L)] = plsc.load_gather(src_sp, [ivec])
                return carry

            lax.fori_loop(0, _OUTW // (8 * _L), step, 0)
            pltpu.sync_copy(out_sp, out_hbm.at[b])

    return gather_k(pidx, xe2d)


def _gelu(v):
    return 0.5 * v * (1.0 + jax.lax.erf(v * np.float32(0.7071067811865476)))


def _pre_body(x_ref, ln1_g, ln1_b, wn, bn, we2p, xf_ref, gn2_ref):
    xb = x_ref[0]
    m = jnp.mean(xb, axis=1, keepdims=True)
    xc = xb - m
    v = jnp.mean(xc * xc, axis=1, keepdims=True)
    xn = xc * jax.lax.rsqrt(v + 1e-5) * ln1_g[0] + ln1_b[0]
    xf = _gelu(jnp.dot(xn, wn[...], preferred_element_type=jnp.float32) + bn[0])
    xf_ref[0] = xf
    gn2_ref[0] = jnp.dot(xf, we2p[...], preferred_element_type=jnp.float32)


def _main_body(xf_ref, xe_ref, gnc_ref, we1k, be1k, be2r, root, cb,
               wf1, bf1, wf2, bf2, lno_g, lno_b, out_ref, acc_ref):
    ic = pl.program_id(1)
    xf = xf_ref[0]

    xe = xe_ref[0, 0]
    hidc = _gelu(jnp.dot(xe, we1k[...], preferred_element_type=jnp.float32) + be1k[0])
    rr = jax.lax.broadcasted_iota(jnp.int32, (RH, KW), 0) + ic * RH
    cc = jax.lax.broadcasted_iota(jnp.int32, (RH, KW), 1)
    diag = (G * (rr // N) + cc // 64) == (rr % N)
    hidc = jnp.where(diag, 0.0, hidc)

    partial = jnp.zeros((N, H), jnp.float32)
    for gl in range(GH):
        partial = partial + jnp.dot(hidc[gl * N:(gl + 1) * N, :],
                                    gnc_ref[0, 0, gl * KW:(gl + 1) * KW, :],
                                    preferred_element_type=jnp.float32)

    prev = jnp.where(ic == 0, 0.0, acc_ref[...])
    total = prev + partial
    acc_ref[...] = total

    @pl.when(ic == 1)
    def _tail():
        s_all = jnp.sum(xf, axis=0, keepdims=True)
        agg = (total + jnp.dot(s_all - xf, be2r[...],
                               preferred_element_type=jnp.float32)) * np.float32(1.0 / 63.0)
        out = agg + jnp.dot(xf, root[...], preferred_element_type=jnp.float32) + cb[0]
        xo = xf + _gelu(out)
        ff = _gelu(jnp.dot(xo, wf1[...], preferred_element_type=jnp.float32) + bf1[0])
        xo = xo + jnp.dot(ff, wf2[...], preferred_element_type=jnp.float32) + bf2[0]
        mo = jnp.mean(xo, axis=1, keepdims=True)
        xoc = xo - mo
        vo = jnp.mean(xoc * xoc, axis=1, keepdims=True)
        out_ref[0] = xoc * jax.lax.rsqrt(vo + 1e-5) * lno_g[0] + lno_b[0]


def kernel(x, x_edge, ln1_g, ln1_b, Wn, bn, We1, be1, We2, be2, root, cb,
           Wf1, bf1, Wf2, bf2, lno_g, lno_b):
    We2p = We2.reshape(64, H, H).transpose(1, 0, 2).reshape(H, 64 * H)
    be2r = be2.reshape(H, H)
    We1k = jnp.kron(jnp.eye(G, dtype=We1.dtype), We1)
    be1k = jnp.tile(be1, G)

    pidx = jnp.asarray(_PIDX)
    xe_perm = _sc_permute(x_edge.reshape(B, EPG * DE), pidx).reshape(B, 2, RH, G * DE)

    row = lambda a: a.reshape(1, -1)

    pre_w = [row(ln1_g), row(ln1_b), Wn, row(bn), We2p]
    fullp = lambda a: pl.BlockSpec(a.shape, lambda b: (0,) * a.ndim)
    xf_all, gn2_all = pl.pallas_call(
        _pre_body,
        grid=(B,),
        in_specs=[pl.BlockSpec((1, N, DIN), lambda b: (b, 0, 0))] + [fullp(w) for w in pre_w],
        out_specs=[pl.BlockSpec((1, N, H), lambda b: (b, 0, 0)),
                   pl.BlockSpec((1, N, 64 * H), lambda b: (b, 0, 0))],
        out_shape=[jax.ShapeDtypeStruct((B, N, H), jnp.float32),
                   jax.ShapeDtypeStruct((B, N, 64 * H), jnp.float32)],
    )(x, *pre_w)

    gnc_all = gn2_all.reshape(B, 2, 2 * GH * KW // 2, H).reshape(B, 2, GH * KW, H)

    main_w = [We1k, row(be1k), be2r, root, row(cb), Wf1, row(bf1), Wf2, row(bf2),
              row(lno_g), row(lno_b)]
    fullm = lambda a: pl.BlockSpec(a.shape, lambda b, ic: (0,) * a.ndim)
    out = pl.pallas_call(
        _main_body,
        grid=(B, 2),
        in_specs=[
            pl.BlockSpec((1, N, H), lambda b, ic: (b, 0, 0)),
            pl.BlockSpec((1, 1, RH, G * DE), lambda b, ic: (b, ic, 0, 0)),
            pl.BlockSpec((1, 1, GH * KW, H), lambda b, ic: (b, ic, 0, 0)),
        ] + [fullm(w) for w in main_w],
        out_specs=pl.BlockSpec((1, N, H), lambda b, ic: (b, 0, 0)),
        out_shape=jax.ShapeDtypeStruct((B, N, H), jnp.float32),
        scratch_shapes=[pltpu.VMEM((N, H), jnp.float32)],
    )(xf_all, xe_perm, gnc_all, *main_w)
    return out


# speedup vs baseline: 8.0882x; 1.3688x over previous
import functools
import numpy as np
import jax
import jax.numpy as jnp
from jax import lax
from jax.experimental import pallas as pl
from jax.experimental.pallas import tpu as pltpu
from jax.experimental.pallas import tpu_sc as plsc

N = 64
B = 64
H = 16
DIN = 128
DE = 4
EPG = 4032
NN = N * N
G = 8
NG = N // G
GH = NG // 2
RH = GH * N
KW = G * 64


def _build_perm():
    tri = np.zeros((N, N), dtype=np.int64)
    iu = np.triu_indices(N, k=1)
    tri[iu] = np.arange(len(iu[0]))
    i = np.arange(N)[:, None]
    j = np.arange(N)[None, :]
    perm2d = np.where(i < j, tri, np.where(i > j, 2016 + tri.T, 0))
    r = np.arange(NN)
    m = r % G
    t = r // G
    jj = t % N
    gg = t // N
    return perm2d[(G * gg + m), jj].astype(np.int32)


_PERM = _build_perm()
_PIDX = (_PERM[:, None].astype(np.int32) * DE + np.arange(DE, dtype=np.int32)[None, :]).reshape(-1)

_NW = 32
_GPW = B // _NW
_SRCW = EPG * DE
_OUTW = NN * DE
_L = 16


def _sc_permute(xe2d, pidx):
    mesh = plsc.VectorSubcoreMesh(core_axis_name="c", subcore_axis_name="s")

    @functools.partial(
        pl.kernel, mesh=mesh,
        out_type=jax.ShapeDtypeStruct((B, _OUTW), jnp.float32),
        compiler_params=pltpu.CompilerParams(use_tc_tiling_on_sc=False,
                                             needs_layout_passes=False),
        scratch_types=[
            pltpu.VMEM((_OUTW,), jnp.int32),
            pltpu.VMEM((_SRCW,), jnp.float32),
            pltpu.VMEM((_OUTW,), jnp.float32),
        ],
    )
    def gather_k(pidx_hbm, src_hbm, out_hbm, idx_sp, src_sp, out_sp):
        wid = lax.axis_index("s") * 2 + lax.axis_index("c")
        pltpu.sync_copy(pidx_hbm, idx_sp)
        for rep in range(_GPW):
            b = wid * _GPW + rep
            pltpu.sync_copy(src_hbm.at[b], src_sp)

            def step(t, carry):
                for u in range(8):
                    v = t * 8 + u
                    ivec = idx_sp[pl.ds(v * _L, _L)]
                    out_sp[pl.ds(v * _L, _L)] = plsc.load_gather(src_sp, [ivec])
                return carry

            lax.fori_loop(0, _OUTW // (8 * _L), step, 0)
            pltpu.sync_copy(out_sp, out_hbm.at[b])

    return gather_k(pidx, xe2d)


def _gelu(v):
    return 0.5 * v * (1.0 + jax.lax.erf(v * np.float32(0.7071067811865476)))


_PB = 8


def _pre_body(x_ref, ln1_g, ln1_b, wn, bn, we2p, xf_ref, gn2_ref):
    xb = x_ref[...].reshape(_PB * N, DIN)
    m = jnp.mean(xb, axis=1, keepdims=True)
    xc = xb - m
    v = jnp.mean(xc * xc, axis=1, keepdims=True)
    xn = xc * jax.lax.rsqrt(v + 1e-5) * ln1_g[0] + ln1_b[0]
    xf = _gelu(jnp.dot(xn, wn[...], preferred_element_type=jnp.float32) + bn[0])
    xf_ref[...] = xf.reshape(_PB, N, H)
    gn2 = jnp.dot(xf, we2p[...], preferred_element_type=jnp.float32)
    gn2_ref[...] = gn2.reshape(_PB, N, 64 * H)


def _main_body(xf_ref, xe_ref, gnc_ref, we1k, be1k, be2r, root, cb,
               wf1, bf1, wf2, bf2, lno_g, lno_b, out_ref):
    xf = xf_ref[0]

    xe = xe_ref[0]
    hidc = _gelu(jnp.dot(xe, we1k[...], preferred_element_type=jnp.float32) + be1k[0])
    rr = jax.lax.broadcasted_iota(jnp.int32, (NG * N, KW), 0)
    cc = jax.lax.broadcasted_iota(jnp.int32, (NG * N, KW), 1)
    diag = (G * (rr // N) + cc // 64) == (rr % N)
    hidc = jnp.where(diag, 0.0, hidc)

    total = jnp.zeros((N, H), jnp.float32)
    for gl in range(NG):
        total = total + jnp.dot(hidc[gl * N:(gl + 1) * N, :],
                                gnc_ref[0, gl * KW:(gl + 1) * KW, :],
                                preferred_element_type=jnp.float32)

    s_all = jnp.sum(xf, axis=0, keepdims=True)
    agg = (total + jnp.dot(s_all - xf, be2r[...],
                           preferred_element_type=jnp.float32)) * np.float32(1.0 / 63.0)
    out = agg + jnp.dot(xf, root[...], preferred_element_type=jnp.float32) + cb[0]
    xo = xf + _gelu(out)
    ff = _gelu(jnp.dot(xo, wf1[...], preferred_element_type=jnp.float32) + bf1[0])
    xo = xo + jnp.dot(ff, wf2[...], preferred_element_type=jnp.float32) + bf2[0]
    mo = jnp.mean(xo, axis=1, keepdims=True)
    xoc = xo - mo
    vo = jnp.mean(xoc * xoc, axis=1, keepdims=True)
    out_ref[0] = xoc * jax.lax.rsqrt(vo + 1e-5) * lno_g[0] + lno_b[0]


def kernel(x, x_edge, ln1_g, ln1_b, Wn, bn, We1, be1, We2, be2, root, cb,
           Wf1, bf1, Wf2, bf2, lno_g, lno_b):
    We2p = We2.reshape(64, H, H).transpose(1, 0, 2).reshape(H, 64 * H)
    be2r = be2.reshape(H, H)
    We1k = jnp.kron(jnp.eye(G, dtype=We1.dtype), We1)
    be1k = jnp.tile(be1, G)

    pidx = jnp.asarray(_PIDX)
    xe_perm = _sc_permute(x_edge.reshape(B, EPG * DE), pidx).reshape(B, NG * N, G * DE)

    row = lambda a: a.reshape(1, -1)

    pre_w = [row(ln1_g), row(ln1_b), Wn, row(bn), We2p]
    fullp = lambda a: pl.BlockSpec(a.shape, lambda b: (0,) * a.ndim)
    xf_all, gn2_all = pl.pallas_call(
        _pre_body,
        grid=(B // _PB,),
        in_specs=[pl.BlockSpec((_PB, N, DIN), lambda b: (b, 0, 0))] + [fullp(w) for w in pre_w],
        out_specs=[pl.BlockSpec((_PB, N, H), lambda b: (b, 0, 0)),
                   pl.BlockSpec((_PB, N, 64 * H), lambda b: (b, 0, 0))],
        out_shape=[jax.ShapeDtypeStruct((B, N, H), jnp.float32),
                   jax.ShapeDtypeStruct((B, N, 64 * H), jnp.float32)],
    )(x, *pre_w)

    gnc_all = gn2_all.reshape(B, NG * KW, H)

    main_w = [We1k, row(be1k), be2r, root, row(cb), Wf1, row(bf1), Wf2, row(bf2),
              row(lno_g), row(lno_b)]
    fullm = lambda a: pl.BlockSpec(a.shape, lambda b: (0,) * a.ndim)
    out = pl.pallas_call(
        _main_body,
        grid=(B,),
        in_specs=[
            pl.BlockSpec((1, N, H), lambda b: (b, 0, 0)),
            pl.BlockSpec((1, NG * N, G * DE), lambda b: (b, 0, 0)),
            pl.BlockSpec((1, NG * KW, H), lambda b: (b, 0, 0)),
        ] + [fullm(w) for w in main_w],
        out_specs=pl.BlockSpec((1, N, H), lambda b: (b, 0, 0)),
        out_shape=jax.ShapeDtypeStruct((B, N, H), jnp.float32),
    )(xf_all, xe_perm, gnc_all, *main_w)
    return out


# speedup vs baseline: 8.7950x; 1.0874x over previous
import functools
import numpy as np
import jax
import jax.numpy as jnp
from jax import lax
from jax.experimental import pallas as pl
from jax.experimental.pallas import tpu as pltpu
from jax.experimental.pallas import tpu_sc as plsc

N = 64
B = 64
H = 16
DIN = 128
DE = 4
EPG = 4032
NN = N * N
G = 8
NG = N // G
GH = NG // 2
RH = GH * N
KW = G * 64


def _build_perm():
    tri = np.zeros((N, N), dtype=np.int64)
    iu = np.triu_indices(N, k=1)
    tri[iu] = np.arange(len(iu[0]))
    i = np.arange(N)[:, None]
    j = np.arange(N)[None, :]
    perm2d = np.where(i < j, tri, np.where(i > j, 2016 + tri.T, 0))
    r = np.arange(NN)
    m = r % G
    t = r // G
    jj = t % N
    gg = t // N
    return perm2d[(G * gg + m), jj].astype(np.int32)


_PERM = _build_perm()
_PIDX = (_PERM[:, None].astype(np.int32) * DE + np.arange(DE, dtype=np.int32)[None, :]).reshape(-1)

_NW = 32
_GPW = B // _NW
_SRCW = EPG * DE
_OUTW = NN * DE
_L = 16


def _sc_permute(xe2d, pidx):
    mesh = plsc.VectorSubcoreMesh(core_axis_name="c", subcore_axis_name="s")

    @functools.partial(
        pl.kernel, mesh=mesh,
        out_type=jax.ShapeDtypeStruct((B, _OUTW), jnp.float32),
        compiler_params=pltpu.CompilerParams(use_tc_tiling_on_sc=False,
                                             needs_layout_passes=False),
        scratch_types=[
            pltpu.VMEM((_OUTW,), jnp.int32),
            pltpu.VMEM((_SRCW,), jnp.float32),
            pltpu.VMEM((_OUTW,), jnp.float32),
        ],
    )
    def gather_k(pidx_hbm, src_hbm, out_hbm, idx_sp, src_sp, out_sp):
        wid = lax.axis_index("s") * 2 + lax.axis_index("c")
        pltpu.sync_copy(pidx_hbm, idx_sp)
        for rep in range(_GPW):
            b = wid * _GPW + rep
            pltpu.sync_copy(src_hbm.at[b], src_sp)

            def step(t, carry):
                for u in range(8):
                    v = t * 8 + u
                    ivec = idx_sp[pl.ds(v * _L, _L)]
                    out_sp[pl.ds(v * _L, _L)] = plsc.load_gather(src_sp, [ivec])
                return carry

            lax.fori_loop(0, _OUTW // (8 * _L), step, 0)
            pltpu.sync_copy(out_sp, out_hbm.at[b])

    return gather_k(pidx, xe2d)


def _gelu(v):
    return 0.5 * v * (1.0 + jax.lax.erf(v * np.float32(0.7071067811865476)))


_PB = 8


def _pre_body(x_ref, ln1_g, ln1_b, wn, bn, we2p, xf_ref, gn2_ref):
    xb = x_ref[...].reshape(_PB * N, DIN)
    m = jnp.mean(xb, axis=1, keepdims=True)
    xc = xb - m
    v = jnp.mean(xc * xc, axis=1, keepdims=True)
    xn = xc * jax.lax.rsqrt(v + 1e-5) * ln1_g[0] + ln1_b[0]
    xf = _gelu(jnp.dot(xn, wn[...], preferred_element_type=jnp.float32) + bn[0])
    xf_ref[...] = xf.reshape(_PB, N, H)
    gn2 = jnp.dot(xf, we2p[...], preferred_element_type=jnp.float32)
    gn2_ref[...] = gn2.reshape(_PB, N, 64 * H)


_MB = 2


def _main_body(xf_ref, xe_ref, gnc_ref, we1k, be1k, be2r, root, cb,
               wf1, bf1, wf2, bf2, lno_g, lno_b, out_ref):
    xe = xe_ref[...].reshape(_MB * NG * N, G * DE)
    hidc2 = _gelu(jnp.dot(xe, we1k[...], preferred_element_type=jnp.float32) + be1k[0])
    rr = jax.lax.broadcasted_iota(jnp.int32, (_MB * NG * N, KW), 0)
    cc = jax.lax.broadcasted_iota(jnp.int32, (_MB * NG * N, KW), 1)
    diag = (G * ((rr % (NG * N)) // N) + cc // 64) == (rr % N)
    hidc2 = jnp.where(diag, 0.0, hidc2)

    for gi in range(_MB):
        xf = xf_ref[gi]
        total = jnp.zeros((N, H), jnp.float32)
        for gl in range(NG):
            total = total + jnp.dot(hidc2[(gi * NG + gl) * N:(gi * NG + gl + 1) * N, :],
                                    gnc_ref[gi, gl * KW:(gl + 1) * KW, :],
                                    preferred_element_type=jnp.float32)

        s_all = jnp.sum(xf, axis=0, keepdims=True)
        agg = (total + jnp.dot(s_all - xf, be2r[...],
                               preferred_element_type=jnp.float32)) * np.float32(1.0 / 63.0)
        out = agg + jnp.dot(xf, root[...], preferred_element_type=jnp.float32) + cb[0]
        xo = xf + _gelu(out)
        ff = _gelu(jnp.dot(xo, wf1[...], preferred_element_type=jnp.float32) + bf1[0])
        xo = xo + jnp.dot(ff, wf2[...], preferred_element_type=jnp.float32) + bf2[0]
        mo = jnp.mean(xo, axis=1, keepdims=True)
        xoc = xo - mo
        vo = jnp.mean(xoc * xoc, axis=1, keepdims=True)
        out_ref[gi] = xoc * jax.lax.rsqrt(vo + 1e-5) * lno_g[0] + lno_b[0]


def kernel(x, x_edge, ln1_g, ln1_b, Wn, bn, We1, be1, We2, be2, root, cb,
           Wf1, bf1, Wf2, bf2, lno_g, lno_b):
    We2p = We2.reshape(64, H, H).transpose(1, 0, 2).reshape(H, 64 * H)
    be2r = be2.reshape(H, H)
    We1k = jnp.kron(jnp.eye(G, dtype=We1.dtype), We1)
    be1k = jnp.tile(be1, G)

    pidx = jnp.asarray(_PIDX)
    xe_perm = _sc_permute(x_edge.reshape(B, EPG * DE), pidx).reshape(B, NG * N, G * DE)

    row = lambda a: a.reshape(1, -1)

    pre_w = [row(ln1_g), row(ln1_b), Wn, row(bn), We2p]
    fullp = lambda a: pl.BlockSpec(a.shape, lambda b: (0,) * a.ndim)
    xf_all, gn2_all = pl.pallas_call(
        _pre_body,
        grid=(B // _PB,),
        in_specs=[pl.BlockSpec((_PB, N, DIN), lambda b: (b, 0, 0))] + [fullp(w) for w in pre_w],
        out_specs=[pl.BlockSpec((_PB, N, H), lambda b: (b, 0, 0)),
                   pl.BlockSpec((_PB, N, 64 * H), lambda b: (b, 0, 0))],
        out_shape=[jax.ShapeDtypeStruct((B, N, H), jnp.float32),
                   jax.ShapeDtypeStruct((B, N, 64 * H), jnp.float32)],
    )(x, *pre_w)

    gnc_all = gn2_all.reshape(B, NG * KW, H)

    main_w = [We1k, row(be1k), be2r, root, row(cb), Wf1, row(bf1), Wf2, row(bf2),
              row(lno_g), row(lno_b)]
    fullm = lambda a: pl.BlockSpec(a.shape, lambda b: (0,) * a.ndim)
    out = pl.pallas_call(
        _main_body,
        grid=(B // _MB,),
        in_specs=[
            pl.BlockSpec((_MB, N, H), lambda b: (b, 0, 0)),
            pl.BlockSpec((_MB, NG * N, G * DE), lambda b: (b, 0, 0)),
            pl.BlockSpec((_MB, NG * KW, H), lambda b: (b, 0, 0)),
        ] + [fullm(w) for w in main_w],
        out_specs=pl.BlockSpec((_MB, N, H), lambda b: (b, 0, 0)),
        out_shape=jax.ShapeDtypeStruct((B, N, H), jnp.float32),
    )(xf_all, xe_perm, gnc_all, *main_w)
    return out


# speedup vs baseline: 9.2080x; 1.0470x over previous
import functools
import numpy as np
import jax
import jax.numpy as jnp
from jax import lax
from jax.experimental import pallas as pl
from jax.experimental.pallas import tpu as pltpu
from jax.experimental.pallas import tpu_sc as plsc

N = 64
B = 64
H = 16
DIN = 128
DE = 4
EPG = 4032
NN = N * N
G = 8
NG = N // G
GH = NG // 2
RH = GH * N
KW = G * 64


def _build_perm():
    tri = np.zeros((N, N), dtype=np.int64)
    iu = np.triu_indices(N, k=1)
    tri[iu] = np.arange(len(iu[0]))
    i = np.arange(N)[:, None]
    j = np.arange(N)[None, :]
    perm2d = np.where(i < j, tri, np.where(i > j, 2016 + tri.T, 0))
    r = np.arange(NN)
    m = r % G
    t = r // G
    jj = t % N
    gg = t // N
    return perm2d[(G * gg + m), jj].astype(np.int32)


_PERM = _build_perm()
_PIDX = (_PERM[:, None].astype(np.int32) * DE + np.arange(DE, dtype=np.int32)[None, :]).reshape(-1)

_NW = 32
_GPW = B // _NW
_SRCW = EPG * DE
_OUTW = NN * DE
_L = 16


def _sc_permute(xe2d, pidx):
    mesh = plsc.VectorSubcoreMesh(core_axis_name="c", subcore_axis_name="s")

    @functools.partial(
        pl.kernel, mesh=mesh,
        out_type=jax.ShapeDtypeStruct((B, _OUTW), jnp.float32),
        compiler_params=pltpu.CompilerParams(use_tc_tiling_on_sc=False,
                                             needs_layout_passes=False),
        scratch_types=[
            pltpu.VMEM((_OUTW,), jnp.int32),
            pltpu.VMEM((_SRCW,), jnp.float32),
            pltpu.VMEM((_OUTW,), jnp.float32),
        ],
    )
    def gather_k(pidx_hbm, src_hbm, out_hbm, idx_sp, src_sp, out_sp):
        wid = lax.axis_index("s") * 2 + lax.axis_index("c")
        pltpu.sync_copy(pidx_hbm, idx_sp)
        for rep in range(_GPW):
            b = wid * _GPW + rep
            pltpu.sync_copy(src_hbm.at[b], src_sp)

            def step(t, carry):
                for u in range(8):
                    v = t * 8 + u
                    ivec = idx_sp[pl.ds(v * _L, _L)]
                    out_sp[pl.ds(v * _L, _L)] = plsc.load_gather(src_sp, [ivec])
                return carry

            lax.fori_loop(0, _OUTW // (8 * _L), step, 0)
            pltpu.sync_copy(out_sp, out_hbm.at[b])

    return gather_k(pidx, xe2d)


def _gelu(v):
    return 0.5 * v * (1.0 + jax.lax.erf(v * np.float32(0.7071067811865476)))


_PB = 8


def _pre_body(x_ref, ln1_g, ln1_b, wn, bn, we2p, xf_ref, gn2_ref):
    xb = x_ref[...].reshape(_PB * N, DIN)
    m = jnp.mean(xb, axis=1, keepdims=True)
    xc = xb - m
    v = jnp.mean(xc * xc, axis=1, keepdims=True)
    xn = xc * jax.lax.rsqrt(v + 1e-5) * ln1_g[0] + ln1_b[0]
    xf = _gelu(jnp.dot(xn, wn[...], preferred_element_type=jnp.float32) + bn[0])
    xf_ref[...] = xf.reshape(_PB, N, H)
    gn2 = jnp.dot(xf, we2p[...], preferred_element_type=jnp.float32)
    gn2_ref[...] = gn2.reshape(_PB, N, 64 * H)


_MB = 4


def _main_body(xf_ref, xe_ref, gnc_ref, we1k, be1k, be2r, root, cb,
               wf1, bf1, wf2, bf2, lno_g, lno_b, out_ref):
    xe = xe_ref[...].reshape(_MB * NG * N, G * DE)
    hidc2 = _gelu(jnp.dot(xe, we1k[...], preferred_element_type=jnp.float32) + be1k[0])
    rr = jax.lax.broadcasted_iota(jnp.int32, (_MB * NG * N, KW), 0)
    cc = jax.lax.broadcasted_iota(jnp.int32, (_MB * NG * N, KW), 1)
    diag = (G * ((rr % (NG * N)) // N) + cc // 64) == (rr % N)
    hidc2 = jnp.where(diag, 0.0, hidc2)

    for gi in range(_MB):
        xf = xf_ref[gi]
        total = jnp.zeros((N, H), jnp.float32)
        for gl in range(NG):
            total = total + jnp.dot(hidc2[(gi * NG + gl) * N:(gi * NG + gl + 1) * N, :],
                                    gnc_ref[gi, gl * KW:(gl + 1) * KW, :],
                                    preferred_element_type=jnp.float32)

        s_all = jnp.sum(xf, axis=0, keepdims=True)
        agg = (total + jnp.dot(s_all - xf, be2r[...],
                               preferred_element_type=jnp.float32)) * np.float32(1.0 / 63.0)
        out = agg + jnp.dot(xf, root[...], preferred_element_type=jnp.float32) + cb[0]
        xo = xf + _gelu(out)
        ff = _gelu(jnp.dot(xo, wf1[...], preferred_element_type=jnp.float32) + bf1[0])
        xo = xo + jnp.dot(ff, wf2[...], preferred_element_type=jnp.float32) + bf2[0]
        mo = jnp.mean(xo, axis=1, keepdims=True)
        xoc = xo - mo
        vo = jnp.mean(xoc * xoc, axis=1, keepdims=True)
        out_ref[gi] = xoc * jax.lax.rsqrt(vo + 1e-5) * lno_g[0] + lno_b[0]


def kernel(x, x_edge, ln1_g, ln1_b, Wn, bn, We1, be1, We2, be2, root, cb,
           Wf1, bf1, Wf2, bf2, lno_g, lno_b):
    We2p = We2.reshape(64, H, H).transpose(1, 0, 2).reshape(H, 64 * H)
    be2r = be2.reshape(H, H)
    We1k = jnp.kron(jnp.eye(G, dtype=We1.dtype), We1)
    be1k = jnp.tile(be1, G)

    pidx = jnp.asarray(_PIDX)
    xe_perm = _sc_permute(x_edge.reshape(B, EPG * DE), pidx).reshape(B, NG * N, G * DE)

    row = lambda a: a.reshape(1, -1)

    pre_w = [row(ln1_g), row(ln1_b), Wn, row(bn), We2p]
    fullp = lambda a: pl.BlockSpec(a.shape, lambda b: (0,) * a.ndim)
    xf_all, gn2_all = pl.pallas_call(
        _pre_body,
        grid=(B // _PB,),
        in_specs=[pl.BlockSpec((_PB, N, DIN), lambda b: (b, 0, 0))] + [fullp(w) for w in pre_w],
        out_specs=[pl.BlockSpec((_PB, N, H), lambda b: (b, 0, 0)),
                   pl.BlockSpec((_PB, N, 64 * H), lambda b: (b, 0, 0))],
        out_shape=[jax.ShapeDtypeStruct((B, N, H), jnp.float32),
                   jax.ShapeDtypeStruct((B, N, 64 * H), jnp.float32)],
    )(x, *pre_w)

    gnc_all = gn2_all.reshape(B, NG * KW, H)

    main_w = [We1k, row(be1k), be2r, root, row(cb), Wf1, row(bf1), Wf2, row(bf2),
              row(lno_g), row(lno_b)]
    fullm = lambda a: pl.BlockSpec(a.shape, lambda b: (0,) * a.ndim)
    out = pl.pallas_call(
        _main_body,
        grid=(B // _MB,),
        in_specs=[
            pl.BlockSpec((_MB, N, H), lambda b: (b, 0, 0)),
            pl.BlockSpec((_MB, NG * N, G * DE), lambda b: (b, 0, 0)),
            pl.BlockSpec((_MB, NG * KW, H), lambda b: (b, 0, 0)),
        ] + [fullm(w) for w in main_w],
        out_specs=pl.BlockSpec((_MB, N, H), lambda b: (b, 0, 0)),
        out_shape=jax.ShapeDtypeStruct((B, N, H), jnp.float32),
    )(xf_all, xe_perm, gnc_all, *main_w)
    return out
